# baseline (device time: 24536 ns/iter reference)
import jax
import jax.numpy as jnp
from jax import lax
from jax.experimental import pallas as pl
from jax.experimental.pallas import tpu as pltpu


def kernel(O, Wo):
    b, s, h, d = O.shape
    k = h * d
    n = Wo.shape[1]
    s_half = s // 2
    n_half = n // 2

    Ot = jnp.transpose(O, (0, 2, 3, 1))

    def body(ot_hbm, w_hbm, out_hbm,
             ot_ref, w_ref, out_ref, send_ref, xrecv_ref, yrecv_ref,
             load_sems, dsend_sems, xrecv_sems, fsend_sems, yrecv_sems,
             store_sems):
        my_x = lax.axis_index("x")
        my_y = lax.axis_index("y")
        nbr_x = 1 - my_x
        nbr_y = 1 - my_y

        w_load = pltpu.make_async_copy(w_hbm, w_ref, load_sems.at[0])
        w_load.start()
        o_loads = []
        for bi in range(b):
            ld = pltpu.make_async_copy(
                ot_hbm.at[bi], ot_ref.at[bi], load_sems.at[1 + bi]
            )
            ld.start()
            o_loads.append(ld)

        barrier = pltpu.get_barrier_semaphore()
        for dev in [(nbr_x, my_y), (my_x, nbr_y), (nbr_x, nbr_y)]:
            pl.semaphore_signal(
                barrier, inc=1,
                device_id=dev, device_id_type=pl.DeviceIdType.MESH,
            )
        pl.semaphore_wait(barrier, 3)

        w_load.wait()
        w_bf = w_ref[...].astype(jnp.bfloat16)
        w_half = w_ref[:, pl.ds(my_y * n_half, n_half)].astype(jnp.bfloat16)

        def lhs_for(bi, s_start):
            blk = ot_ref[bi, :, :, pl.ds(s_start, s_half)]
            return blk.reshape(k, s_half).astype(jnp.bfloat16)

        directs = []
        for bi in range(b):
            o_loads[bi].wait()
            p = lax.dot_general(
                lhs_for(bi, nbr_x * s_half), w_half,
                (((0,), (0,)), ((), ())),
                preferred_element_type=jnp.float32,
            )
            send_ref[bi] = p.astype(jnp.bfloat16)
            r = pltpu.make_async_remote_copy(
                src_ref=send_ref.at[bi],
                dst_ref=xrecv_ref.at[bi],
                send_sem=dsend_sems.at[bi],
                recv_sem=xrecv_sems.at[bi],
                device_id=(nbr_x, my_y),
                device_id_type=pl.DeviceIdType.MESH,
            )
            r.start()
            directs.append(r)

        forwards = []
        for bi in range(b):
            out_ref[bi] = lax.dot_general(
                lhs_for(bi, my_x * s_half), w_bf,
                (((0,), (0,)), ((), ())),
                preferred_element_type=jnp.float32,
            )
            directs[bi].wait()
            f = pltpu.make_async_remote_copy(
                src_ref=xrecv_ref.at[bi],
                dst_ref=yrecv_ref.at[bi],
                send_sem=fsend_sems.at[bi],
                recv_sem=yrecv_sems.at[bi],
                device_id=(my_x, nbr_y),
                device_id_type=pl.DeviceIdType.MESH,
            )
            f.start()
            forwards.append(f)
            out_ref[bi, :, pl.ds(my_y * n_half, n_half)] = (
                out_ref[bi, :, pl.ds(my_y * n_half, n_half)]
                + xrecv_ref[bi].astype(jnp.float32)
            )

        stores = []
        for bi in range(b):
            forwards[bi].wait()
            out_ref[bi, :, pl.ds(nbr_y * n_half, n_half)] = (
                out_ref[bi, :, pl.ds(nbr_y * n_half, n_half)]
                + yrecv_ref[bi].astype(jnp.float32)
            )
            st = pltpu.make_async_copy(
                out_ref.at[bi], out_hbm.at[bi], store_sems.at[bi]
            )
            st.start()
            stores.append(st)
        for st in stores:
            st.wait()

    return pl.pallas_call(
        body,
        out_shape=jax.ShapeDtypeStruct((b, s_half, n), jnp.float32),
        in_specs=[
            pl.BlockSpec(memory_space=pltpu.MemorySpace.HBM),
            pl.BlockSpec(memory_space=pltpu.MemorySpace.HBM),
        ],
        out_specs=pl.BlockSpec(memory_space=pltpu.MemorySpace.HBM),
        scratch_shapes=[
            pltpu.VMEM((b, h, d, s), jnp.float32),
            pltpu.VMEM((k, n), jnp.float32),
            pltpu.VMEM((b, s_half, n), jnp.float32),
            pltpu.VMEM((b, s_half, n_half), jnp.bfloat16),
            pltpu.VMEM((b, s_half, n_half), jnp.bfloat16),
            pltpu.VMEM((b, s_half, n_half), jnp.bfloat16),
            pltpu.SemaphoreType.DMA((1 + b,)),
            pltpu.SemaphoreType.DMA((b,)),
            pltpu.SemaphoreType.DMA((b,)),
            pltpu.SemaphoreType.DMA((b,)),
            pltpu.SemaphoreType.DMA((b,)),
            pltpu.SemaphoreType.DMA((b,)),
        ],
        compiler_params=pltpu.CompilerParams(collective_id=0),
    )(
        pltpu.with_memory_space_constraint(Ot, pltpu.MemorySpace.HBM),
        pltpu.with_memory_space_constraint(Wo, pltpu.MemorySpace.HBM),
    )


# device time: 23172 ns/iter; 1.0589x vs baseline; 1.0589x over previous
import jax
import jax.numpy as jnp
from jax import lax
from jax.experimental import pallas as pl
from jax.experimental.pallas import tpu as pltpu


def kernel(O, Wo):
    b, s, h, d = O.shape
    k = h * d
    n = Wo.shape[1]
    s_half = s // 2
    n_half = n // 2

    Ot = jnp.transpose(O, (0, 2, 3, 1))

    s_q = s_half // 2
    n_chunks = 2 * b

    def body(ot_hbm, w_hbm, out_hbm,
             ot_ref, w_my_ref, w_oth_ref, out_ref,
             send_ref, xrecv_ref, yrecv_ref,
             load_sems, dsend_sems, xrecv_sems, fsend_sems, yrecv_sems,
             store_sems):
        my_x = lax.axis_index("x")
        my_y = lax.axis_index("y")
        nbr_x = 1 - my_x
        nbr_y = 1 - my_y

        w_my_load = pltpu.make_async_copy(
            w_hbm.at[:, pl.ds(my_y * n_half, n_half)], w_my_ref,
            load_sems.at[0],
        )
        w_my_load.start()
        w_oth_load = pltpu.make_async_copy(
            w_hbm.at[:, pl.ds(nbr_y * n_half, n_half)], w_oth_ref,
            load_sems.at[1],
        )
        w_oth_load.start()
        o_loads = []
        for bi in range(b):
            ld = pltpu.make_async_copy(
                ot_hbm.at[bi], ot_ref.at[bi], load_sems.at[2 + bi]
            )
            ld.start()
            o_loads.append(ld)

        barrier = pltpu.get_barrier_semaphore()
        for dev in [(nbr_x, my_y), (my_x, nbr_y), (nbr_x, nbr_y)]:
            pl.semaphore_signal(
                barrier, inc=1,
                device_id=dev, device_id_type=pl.DeviceIdType.MESH,
            )
        pl.semaphore_wait(barrier, 3)

        w_my_load.wait()
        w_my = w_my_ref[...].astype(jnp.bfloat16)

        directs = []
        for ci in range(n_chunks):
            bi, si = ci // 2, ci % 2
            if si == 0:
                o_loads[bi].wait()
            lhs = ot_ref[
                bi, :, :, pl.ds(nbr_x * s_half + si * s_q, s_q)
            ].reshape(k, s_q).astype(jnp.bfloat16)
            p = lax.dot_general(
                lhs, w_my, (((0,), (0,)), ((), ())),
                preferred_element_type=jnp.float32,
            )
            send_ref[ci] = p.astype(jnp.bfloat16)
            r = pltpu.make_async_remote_copy(
                src_ref=send_ref.at[ci],
                dst_ref=xrecv_ref.at[ci],
                send_sem=dsend_sems.at[ci],
                recv_sem=xrecv_sems.at[ci],
                device_id=(nbr_x, my_y),
                device_id_type=pl.DeviceIdType.MESH,
            )
            r.start()
            directs.append(r)

        w_oth_load.wait()
        w_oth = w_oth_ref[...].astype(jnp.bfloat16)
        forwards = []
        for bi in range(b):
            lhs_own = ot_ref[
                bi, :, :, pl.ds(my_x * s_half, s_half)
            ].reshape(k, s_half).astype(jnp.bfloat16)
            out_ref[bi, :, pl.ds(my_y * n_half, n_half)] = lax.dot_general(
                lhs_own, w_my, (((0,), (0,)), ((), ())),
                preferred_element_type=jnp.float32,
            )
            out_ref[bi, :, pl.ds(nbr_y * n_half, n_half)] = lax.dot_general(
                lhs_own, w_oth, (((0,), (0,)), ((), ())),
                preferred_element_type=jnp.float32,
            )
            for si in range(2):
                ci = 2 * bi + si
                directs[ci].wait()
                f = pltpu.make_async_remote_copy(
                    src_ref=xrecv_ref.at[ci],
                    dst_ref=yrecv_ref.at[ci],
                    send_sem=fsend_sems.at[ci],
                    recv_sem=yrecv_sems.at[ci],
                    device_id=(my_x, nbr_y),
                    device_id_type=pl.DeviceIdType.MESH,
                )
                f.start()
                forwards.append(f)
                rows = pl.ds(si * s_q, s_q)
                out_ref[bi, rows, pl.ds(my_y * n_half, n_half)] = (
                    out_ref[bi, rows, pl.ds(my_y * n_half, n_half)]
                    + xrecv_ref[ci].astype(jnp.float32)
                )

        stores = []
        for bi in range(b):
            for si in range(2):
                ci = 2 * bi + si
                forwards[ci].wait()
                rows = pl.ds(si * s_q, s_q)
                out_ref[bi, rows, pl.ds(nbr_y * n_half, n_half)] = (
                    out_ref[bi, rows, pl.ds(nbr_y * n_half, n_half)]
                    + yrecv_ref[ci].astype(jnp.float32)
                )
            st = pltpu.make_async_copy(
                out_ref.at[bi], out_hbm.at[bi], store_sems.at[bi]
            )
            st.start()
            stores.append(st)
        for st in stores:
            st.wait()

    return pl.pallas_call(
        body,
        out_shape=jax.ShapeDtypeStruct((b, s_half, n), jnp.float32),
        in_specs=[
            pl.BlockSpec(memory_space=pltpu.MemorySpace.HBM),
            pl.BlockSpec(memory_space=pltpu.MemorySpace.HBM),
        ],
        out_specs=pl.BlockSpec(memory_space=pl.ANY),
        scratch_shapes=[
            pltpu.VMEM((b, h, d, s), jnp.float32),
            pltpu.VMEM((k, n_half), jnp.float32),
            pltpu.VMEM((k, n_half), jnp.float32),
            pltpu.VMEM((b, s_half, n), jnp.float32),
            pltpu.VMEM((n_chunks, s_q, n_half), jnp.bfloat16),
            pltpu.VMEM((n_chunks, s_q, n_half), jnp.bfloat16),
            pltpu.VMEM((n_chunks, s_q, n_half), jnp.bfloat16),
            pltpu.SemaphoreType.DMA((2 + b,)),
            pltpu.SemaphoreType.DMA((n_chunks,)),
            pltpu.SemaphoreType.DMA((n_chunks,)),
            pltpu.SemaphoreType.DMA((n_chunks,)),
            pltpu.SemaphoreType.DMA((n_chunks,)),
            pltpu.SemaphoreType.DMA((b,)),
        ],
        compiler_params=pltpu.CompilerParams(collective_id=0),
    )(
        pltpu.with_memory_space_constraint(Ot, pltpu.MemorySpace.HBM),
        pltpu.with_memory_space_constraint(Wo, pltpu.MemorySpace.HBM),
    )
